# baseline (speedup 1.0000x reference)
import jax
import jax.numpy as jnp
from jax.experimental import pallas as pl
from jax.experimental.pallas import tpu as pltpu

_B, _N, _F, _K = 8, 4096, 64, 8
_RBLK = 1024
_CHUNK = 1024
_UNROLL = 8


def _scores_body(a_ref, bfull_ref, out_ref, eT_ref):
    a = a_ref[0]
    m = None
    for c in range(_N // _CHUNK):
        bc = bfull_ref[0, _CHUNK * c:_CHUNK * (c + 1), :]
        simc = jax.lax.dot_general(bc, a, (((1,), (1,)), ((), ())),
                                   precision="default")
        eT_ref[_CHUNK * c:_CHUNK * (c + 1), :] = simc
        mc = jnp.max(simc, axis=0, keepdims=True)
        m = mc if m is None else jnp.maximum(m, mc)

    def chain_s(g, acc):
        base = _UNROLL * 8 * g
        for k in range(_UNROLL):
            sl = pl.ds(base + 8 * k, 8)
            t = jnp.exp(eT_ref[sl, :] - m)
            eT_ref[sl, :] = t
            acc = acc + t
        return acc

    acc = jnp.exp(eT_ref[0:8, :] - m)
    eT_ref[0:8, :] = acc
    for k in range(1, _UNROLL):
        t = jnp.exp(eT_ref[8 * k:8 * k + 8, :] - m)
        eT_ref[8 * k:8 * k + 8, :] = t
        acc = acc + t
    acc = jax.lax.fori_loop(1, _N // (8 * _UNROLL), chain_s, acc)
    y = jnp.concatenate([acc[4:8], acc[0:4]], 0) + acc
    z = jnp.concatenate([y[2:8], y[0:2]], 0) + y
    w = jnp.concatenate([z[1:8], z[0:1]], 0) + z

    q = jnp.float32(1.0) / w

    def chain_p(g, acc):
        base = _UNROLL * 8 * g
        for k in range(_UNROLL):
            acc = acc + eT_ref[pl.ds(base + 8 * k, 8), :] * q
        return acc

    acc2 = eT_ref[0:8, :] * q
    for k in range(1, _UNROLL):
        acc2 = acc2 + eT_ref[8 * k:8 * k + 8, :] * q
    acc2 = jax.lax.fori_loop(1, _N // (8 * _UNROLL), chain_p, acc2)
    t = acc2[0:4] + acc2[4:8]
    t = t[0:2] + t[2:4]
    s = t[0:1] + t[1:2]
    out_ref[0, 0] = s[0] * jnp.float32(1.0 / _N)


def _topk_body(scores_ref, feat_ref, hard_ref, conf_ref):
    s = scores_ref[:, 0, :]
    iota = jax.lax.broadcasted_iota(jnp.int32, (_B, _N), 1)

    def select8(vals, out_ref):
        work = vals
        for j in range(_K):
            m = jnp.max(work, axis=1, keepdims=True)
            idx = jnp.min(jnp.where(work == m, iota, _N), axis=1
                          ).astype(jnp.int32)
            for b in range(_B):
                out_ref[b, j, :] = feat_ref[b, pl.ds(idx[b], 1), :][0]
            work = jnp.where(iota == idx[:, None], -jnp.inf, work)

    select8(-s, hard_ref)
    select8(s, conf_ref)


@jax.jit
def kernel(feat):
    norm = jnp.linalg.norm(feat, ord=2, axis=2, keepdims=True)
    normed = feat / norm

    scores = pl.pallas_call(
        _scores_body,
        grid=(_B, _N // _RBLK),
        in_specs=[
            pl.BlockSpec((1, _RBLK, _F), lambda b, i: (b, i, 0)),
            pl.BlockSpec((1, _N, _F), lambda b, i: (b, 0, 0)),
        ],
        out_specs=pl.BlockSpec((1, 1, _RBLK), lambda b, i: (b, 0, i)),
        out_shape=jax.ShapeDtypeStruct((_B, 1, _N), jnp.float32),
        scratch_shapes=[pltpu.VMEM((_N, _RBLK), jnp.float32)],
    )(normed, normed)

    hard, conf = pl.pallas_call(
        _topk_body,
        grid=(1,),
        in_specs=[
            pl.BlockSpec((_B, 1, _N), lambda i: (0, 0, 0)),
            pl.BlockSpec((_B, _N, _F), lambda i: (0, 0, 0)),
        ],
        out_specs=[
            pl.BlockSpec((_B, _K, _F), lambda i: (0, 0, 0)),
            pl.BlockSpec((_B, _K, _F), lambda i: (0, 0, 0)),
        ],
        out_shape=[
            jax.ShapeDtypeStruct((_B, _K, _F), jnp.float32),
            jax.ShapeDtypeStruct((_B, _K, _F), jnp.float32),
        ],
    )(scores, feat)
    return (hard, conf)


# speedup vs baseline: 1.3591x; 1.3591x over previous
import jax
import jax.numpy as jnp
from jax.experimental import pallas as pl
from jax.experimental.pallas import tpu as pltpu

_B, _N, _F, _K = 8, 4096, 64, 8
_RBLK = 1024
_CHUNK = 1024
_UNROLL = 8


def _scores_body(a_ref, bfull_ref, out_ref, eT_ref):
    a = a_ref[0]
    m = None
    for c in range(_N // _CHUNK):
        bc = bfull_ref[0, _CHUNK * c:_CHUNK * (c + 1), :]
        simc = jax.lax.dot_general(bc, a, (((1,), (1,)), ((), ())),
                                   precision="default")
        eT_ref[_CHUNK * c:_CHUNK * (c + 1), :] = simc
        mc = jnp.max(simc, axis=0, keepdims=True)
        m = mc if m is None else jnp.maximum(m, mc)

    for c in range(_N // _CHUNK):
        sl = pl.ds(_CHUNK * c, _CHUNK)
        eT_ref[sl, :] = jnp.exp(eT_ref[sl, :] - m)

    def chain_s(g, acc):
        base = _UNROLL * 8 * g
        for k in range(_UNROLL):
            acc = acc + eT_ref[pl.ds(base + 8 * k, 8), :]
        return acc

    acc = eT_ref[0:8, :]
    for k in range(1, _UNROLL):
        acc = acc + eT_ref[8 * k:8 * k + 8, :]
    acc = jax.lax.fori_loop(1, _N // (8 * _UNROLL), chain_s, acc)
    y = jnp.concatenate([acc[4:8], acc[0:4]], 0) + acc
    z = jnp.concatenate([y[2:8], y[0:2]], 0) + y
    w = jnp.concatenate([z[1:8], z[0:1]], 0) + z

    q = jnp.float32(1.0) / w

    def chain_p(g, acc):
        base = _UNROLL * 8 * g
        for k in range(_UNROLL):
            acc = acc + eT_ref[pl.ds(base + 8 * k, 8), :] * q
        return acc

    acc2 = eT_ref[0:8, :] * q
    for k in range(1, _UNROLL):
        acc2 = acc2 + eT_ref[8 * k:8 * k + 8, :] * q
    acc2 = jax.lax.fori_loop(1, _N // (8 * _UNROLL), chain_p, acc2)
    t = acc2[0:4] + acc2[4:8]
    t = t[0:2] + t[2:4]
    s = t[0:1] + t[1:2]
    out_ref[0, 0] = s[0] * jnp.float32(1.0 / _N)


def _topk_body(scores_ref, feat_ref, hard_ref, conf_ref):
    s = scores_ref[:, 0, :]
    iota = jax.lax.broadcasted_iota(jnp.int32, (_B, _N), 1)

    def select8(vals, out_ref):
        work = vals
        for j in range(_K):
            m = jnp.max(work, axis=1, keepdims=True)
            idx = jnp.min(jnp.where(work == m, iota, _N), axis=1
                          ).astype(jnp.int32)
            for b in range(_B):
                out_ref[b, j, :] = feat_ref[b, pl.ds(idx[b], 1), :][0]
            work = jnp.where(iota == idx[:, None], -jnp.inf, work)

    select8(-s, hard_ref)
    select8(s, conf_ref)


@jax.jit
def kernel(feat):
    norm = jnp.linalg.norm(feat, ord=2, axis=2, keepdims=True)
    normed = feat / norm

    scores = pl.pallas_call(
        _scores_body,
        grid=(_B, _N // _RBLK),
        in_specs=[
            pl.BlockSpec((1, _RBLK, _F), lambda b, i: (b, i, 0)),
            pl.BlockSpec((1, _N, _F), lambda b, i: (b, 0, 0)),
        ],
        out_specs=pl.BlockSpec((1, 1, _RBLK), lambda b, i: (b, 0, i)),
        out_shape=jax.ShapeDtypeStruct((_B, 1, _N), jnp.float32),
        scratch_shapes=[pltpu.VMEM((_N, _RBLK), jnp.float32)],
    )(normed, normed)

    hard, conf = pl.pallas_call(
        _topk_body,
        grid=(1,),
        in_specs=[
            pl.BlockSpec((_B, 1, _N), lambda i: (0, 0, 0)),
            pl.BlockSpec((_B, _N, _F), lambda i: (0, 0, 0)),
        ],
        out_specs=[
            pl.BlockSpec((_B, _K, _F), lambda i: (0, 0, 0)),
            pl.BlockSpec((_B, _K, _F), lambda i: (0, 0, 0)),
        ],
        out_shape=[
            jax.ShapeDtypeStruct((_B, _K, _F), jnp.float32),
            jax.ShapeDtypeStruct((_B, _K, _F), jnp.float32),
        ],
    )(scores, feat)
    return (hard, conf)
